# baseline (speedup 1.0000x reference)
import jax
import jax.numpy as jnp
from jax.experimental import pallas as pl
from jax.experimental.pallas import tpu as pltpu

_B, _N, _M, _C1, _C2 = 4, 8192, 1024, 64, 128
_COUT = 128
_NT = 512
_NB = _N // _NT


def _fp_fwd_kernel(uT_ref, uTb_ref, knb_ref, known_ref, uf_ref, kf_ref,
                   w0_ref, x_ref, sum_ref, ssq_ref):
    uT = uT_ref[0]
    kn = known_ref[0]

    cross = jax.lax.dot(knb_ref[0], uTb_ref[0],
                        preferred_element_type=jnp.float32)
    k2 = jnp.sum(kn * kn, axis=1, keepdims=True)
    u2 = jnp.sum(uT * uT, axis=0, keepdims=True)
    d2 = (u2 + k2) - 2.0 * cross

    inf = jnp.float32(jnp.inf)
    v1 = jnp.min(d2, axis=0, keepdims=True)
    c1 = d2 == v1
    w2 = jnp.where(c1, inf, d2)
    v2 = jnp.min(w2, axis=0, keepdims=True)
    c2 = w2 == v2
    w3 = jnp.where(c2, inf, w2)
    v3 = jnp.min(w3, axis=0, keepdims=True)
    c3 = w3 == v3

    r1, r2, r3 = [1.0 / (jnp.sqrt(jnp.maximum(v, 0.0)) + 1e-8)
                  for v in (v1, v2, v3)]
    inv_norm = 1.0 / (r1 + r2 + r3)
    S = jnp.where(c1, r1 * inv_norm,
                  jnp.where(c2, r2 * inv_norm,
                            jnp.where(c3, r3 * inv_norm, 0.0)))

    interp = jax.lax.dot(kf_ref[0], S, preferred_element_type=jnp.float32)
    w0 = w0_ref[...]
    x = (jax.lax.dot(w0[:, :_C2], interp, preferred_element_type=jnp.float32)
         + jax.lax.dot(w0[:, _C2:], uf_ref[0],
                       preferred_element_type=jnp.float32))
    x_ref[0] = x

    @pl.when(pl.program_id(1) == 0)
    def _init():
        sum_ref[...] = jnp.zeros_like(sum_ref)
        ssq_ref[...] = jnp.zeros_like(ssq_ref)

    sum_ref[0] += jnp.sum(x, axis=1, keepdims=True)
    ssq_ref[0] += jnp.sum(x * x, axis=1, keepdims=True)


def _bn_relu_kernel(x_ref, sum_ref, ssq_ref, g_ref, b_ref, o_ref):
    cnt = jnp.float32(_B * _N)
    mean = jnp.sum(sum_ref[...], axis=0) / cnt
    var = jnp.sum(ssq_ref[...], axis=0) / cnt - mean * mean
    scale = g_ref[...] * jax.lax.rsqrt(var + 1e-5)
    shift = b_ref[...] - mean * scale
    o_ref[0] = jnp.maximum(x_ref[0] * scale + shift, 0.0)


def kernel(unknown, known, unknow_feats, known_feats, W0, gamma0, beta0):
    uT = jnp.transpose(unknown, (0, 2, 1))
    uTb = uT.astype(jnp.bfloat16)
    knb = known.astype(jnp.bfloat16)
    grid = (_B, _NB)
    x, s, ss = pl.pallas_call(
        _fp_fwd_kernel,
        grid=grid,
        in_specs=[
            pl.BlockSpec((1, 3, _NT), lambda b, n: (b, 0, n)),
            pl.BlockSpec((1, 3, _NT), lambda b, n: (b, 0, n)),
            pl.BlockSpec((1, _M, 3), lambda b, n: (b, 0, 0)),
            pl.BlockSpec((1, _M, 3), lambda b, n: (b, 0, 0)),
            pl.BlockSpec((1, _C1, _NT), lambda b, n: (b, 0, n)),
            pl.BlockSpec((1, _C2, _M), lambda b, n: (b, 0, 0)),
            pl.BlockSpec((_COUT, _C1 + _C2), lambda b, n: (0, 0)),
        ],
        out_specs=[
            pl.BlockSpec((1, _COUT, _NT), lambda b, n: (b, 0, n)),
            pl.BlockSpec((1, _COUT, 1), lambda b, n: (b, 0, 0)),
            pl.BlockSpec((1, _COUT, 1), lambda b, n: (b, 0, 0)),
        ],
        out_shape=[
            jax.ShapeDtypeStruct((_B, _COUT, _N), jnp.float32),
            jax.ShapeDtypeStruct((_B, _COUT, 1), jnp.float32),
            jax.ShapeDtypeStruct((_B, _COUT, 1), jnp.float32),
        ],
    )(uT, uTb, knb, known, unknow_feats, known_feats, W0)

    out = pl.pallas_call(
        _bn_relu_kernel,
        grid=grid,
        in_specs=[
            pl.BlockSpec((1, _COUT, _NT), lambda b, n: (b, 0, n)),
            pl.BlockSpec((_B, _COUT, 1), lambda b, n: (0, 0, 0)),
            pl.BlockSpec((_B, _COUT, 1), lambda b, n: (0, 0, 0)),
            pl.BlockSpec((_COUT, 1), lambda b, n: (0, 0)),
            pl.BlockSpec((_COUT, 1), lambda b, n: (0, 0)),
        ],
        out_specs=pl.BlockSpec((1, _COUT, _NT), lambda b, n: (b, 0, n)),
        out_shape=jax.ShapeDtypeStruct((_B, _COUT, _N), jnp.float32),
    )(x, s, ss, gamma0.reshape(_COUT, 1), beta0.reshape(_COUT, 1))
    return out


# speedup vs baseline: 1.0440x; 1.0440x over previous
import jax
import jax.numpy as jnp
from jax.experimental import pallas as pl
from jax.experimental.pallas import tpu as pltpu

_B, _N, _M, _C1, _C2 = 4, 8192, 1024, 64, 128
_COUT = 128
_NT = 512
_NB = _N // _NT


def _fp_fwd_kernel(uT_ref, known_ref, uf_ref, kf_ref,
                   w0_ref, x_ref, sum_ref, ssq_ref):
    uT = uT_ref[0]
    kn = known_ref[0]

    cross = jax.lax.dot(kn.astype(jnp.bfloat16), uT.astype(jnp.bfloat16),
                        preferred_element_type=jnp.float32)
    k2 = jnp.sum(kn * kn, axis=1, keepdims=True)
    u2 = jnp.sum(uT * uT, axis=0, keepdims=True)
    d2 = (u2 + k2) - 2.0 * cross

    inf = jnp.float32(jnp.inf)
    v1 = jnp.min(d2, axis=0, keepdims=True)
    c1 = d2 == v1
    w2 = jnp.where(c1, inf, d2)
    v2 = jnp.min(w2, axis=0, keepdims=True)
    c2 = w2 == v2
    w3 = jnp.where(c2, inf, w2)
    v3 = jnp.min(w3, axis=0, keepdims=True)
    c3 = w3 == v3

    r1, r2, r3 = [1.0 / (jnp.sqrt(jnp.maximum(v, 0.0)) + 1e-8)
                  for v in (v1, v2, v3)]
    inv_norm = 1.0 / (r1 + r2 + r3)
    S = jnp.where(c1, r1 * inv_norm,
                  jnp.where(c2, r2 * inv_norm,
                            jnp.where(c3, r3 * inv_norm, 0.0)))

    interp = jax.lax.dot(kf_ref[0], S, preferred_element_type=jnp.float32)
    w0 = w0_ref[...]
    x = (jax.lax.dot(w0[:, :_C2], interp, preferred_element_type=jnp.float32)
         + jax.lax.dot(w0[:, _C2:], uf_ref[0],
                       preferred_element_type=jnp.float32))
    x_ref[0] = x

    @pl.when(pl.program_id(1) == 0)
    def _init():
        sum_ref[...] = jnp.zeros_like(sum_ref)
        ssq_ref[...] = jnp.zeros_like(ssq_ref)

    sum_ref[0] += jnp.sum(x, axis=1, keepdims=True)
    ssq_ref[0] += jnp.sum(x * x, axis=1, keepdims=True)


def _bn_relu_kernel(x_ref, sum_ref, ssq_ref, g_ref, b_ref, o_ref):
    cnt = jnp.float32(_B * _N)
    mean = jnp.sum(sum_ref[...], axis=0) / cnt
    var = jnp.sum(ssq_ref[...], axis=0) / cnt - mean * mean
    scale = g_ref[...] * jax.lax.rsqrt(var + 1e-5)
    shift = b_ref[...] - mean * scale
    o_ref[0] = jnp.maximum(x_ref[0] * scale + shift, 0.0)


def kernel(unknown, known, unknow_feats, known_feats, W0, gamma0, beta0):
    uT = jnp.transpose(unknown, (0, 2, 1))
    grid = (_B, _NB)
    x, s, ss = pl.pallas_call(
        _fp_fwd_kernel,
        grid=grid,
        in_specs=[
            pl.BlockSpec((1, 3, _NT), lambda b, n: (b, 0, n)),
            pl.BlockSpec((1, _M, 3), lambda b, n: (b, 0, 0)),
            pl.BlockSpec((1, _C1, _NT), lambda b, n: (b, 0, n)),
            pl.BlockSpec((1, _C2, _M), lambda b, n: (b, 0, 0)),
            pl.BlockSpec((_COUT, _C1 + _C2), lambda b, n: (0, 0)),
        ],
        out_specs=[
            pl.BlockSpec((1, _COUT, _NT), lambda b, n: (b, 0, n)),
            pl.BlockSpec((1, _COUT, 1), lambda b, n: (b, 0, 0)),
            pl.BlockSpec((1, _COUT, 1), lambda b, n: (b, 0, 0)),
        ],
        out_shape=[
            jax.ShapeDtypeStruct((_B, _COUT, _N), jnp.float32),
            jax.ShapeDtypeStruct((_B, _COUT, 1), jnp.float32),
            jax.ShapeDtypeStruct((_B, _COUT, 1), jnp.float32),
        ],
    )(uT, known, unknow_feats, known_feats, W0)

    out = pl.pallas_call(
        _bn_relu_kernel,
        grid=grid,
        in_specs=[
            pl.BlockSpec((1, _COUT, _NT), lambda b, n: (b, 0, n)),
            pl.BlockSpec((_B, _COUT, 1), lambda b, n: (0, 0, 0)),
            pl.BlockSpec((_B, _COUT, 1), lambda b, n: (0, 0, 0)),
            pl.BlockSpec((_COUT, 1), lambda b, n: (0, 0)),
            pl.BlockSpec((_COUT, 1), lambda b, n: (0, 0)),
        ],
        out_specs=pl.BlockSpec((1, _COUT, _NT), lambda b, n: (b, 0, n)),
        out_shape=jax.ShapeDtypeStruct((_B, _COUT, _N), jnp.float32),
    )(x, s, ss, gamma0.reshape(_COUT, 1), beta0.reshape(_COUT, 1))
    return out


# speedup vs baseline: 1.2927x; 1.2381x over previous
import jax
import jax.numpy as jnp
from jax.experimental import pallas as pl
from jax.experimental.pallas import tpu as pltpu

_B, _N, _M, _C1, _C2 = 4, 8192, 1024, 64, 128
_COUT = 128
_NT = 1024
_NB = _N // _NT


def _fp_fwd_kernel(uT_ref, known_ref, uf_ref, kf_ref,
                   w0_ref, x_ref, sum_ref, ssq_ref):
    uT = uT_ref[0]
    kn = known_ref[0]

    cross = jax.lax.dot(kn.astype(jnp.bfloat16), uT.astype(jnp.bfloat16),
                        preferred_element_type=jnp.float32)
    k2 = jnp.sum(kn * kn, axis=1, keepdims=True)
    u2 = jnp.sum(uT * uT, axis=0, keepdims=True)
    d2 = (u2 + k2) - 2.0 * cross

    inf = jnp.float32(jnp.inf)
    v1 = jnp.min(d2, axis=0, keepdims=True)
    c1 = d2 == v1
    w2 = jnp.where(c1, inf, d2)
    v2 = jnp.min(w2, axis=0, keepdims=True)
    c2 = w2 == v2
    w3 = jnp.where(c2, inf, w2)
    v3 = jnp.min(w3, axis=0, keepdims=True)
    c3 = w3 == v3

    r1, r2, r3 = [1.0 / (jnp.sqrt(jnp.maximum(v, 0.0)) + 1e-8)
                  for v in (v1, v2, v3)]
    inv_norm = 1.0 / (r1 + r2 + r3)
    S = jnp.where(c1, r1 * inv_norm,
                  jnp.where(c2, r2 * inv_norm,
                            jnp.where(c3, r3 * inv_norm, 0.0)))

    interp = jax.lax.dot(kf_ref[0], S, preferred_element_type=jnp.float32)
    w0 = w0_ref[...]
    x = (jax.lax.dot(w0[:, :_C2], interp, preferred_element_type=jnp.float32)
         + jax.lax.dot(w0[:, _C2:], uf_ref[0],
                       preferred_element_type=jnp.float32))
    x_ref[0] = x

    @pl.when(pl.program_id(1) == 0)
    def _init():
        sum_ref[...] = jnp.zeros_like(sum_ref)
        ssq_ref[...] = jnp.zeros_like(ssq_ref)

    sum_ref[0] += jnp.sum(x, axis=1, keepdims=True)
    ssq_ref[0] += jnp.sum(x * x, axis=1, keepdims=True)


def _bn_relu_kernel(x_ref, sum_ref, ssq_ref, g_ref, b_ref, o_ref):
    cnt = jnp.float32(_B * _N)
    mean = jnp.sum(sum_ref[...], axis=0) / cnt
    var = jnp.sum(ssq_ref[...], axis=0) / cnt - mean * mean
    scale = g_ref[...] * jax.lax.rsqrt(var + 1e-5)
    shift = b_ref[...] - mean * scale
    o_ref[0] = jnp.maximum(x_ref[0] * scale + shift, 0.0)


def kernel(unknown, known, unknow_feats, known_feats, W0, gamma0, beta0):
    uT = jnp.transpose(unknown, (0, 2, 1))
    grid = (_B, _NB)
    x, s, ss = pl.pallas_call(
        _fp_fwd_kernel,
        grid=grid,
        in_specs=[
            pl.BlockSpec((1, 3, _NT), lambda b, n: (b, 0, n)),
            pl.BlockSpec((1, _M, 3), lambda b, n: (b, 0, 0)),
            pl.BlockSpec((1, _C1, _NT), lambda b, n: (b, 0, n)),
            pl.BlockSpec((1, _C2, _M), lambda b, n: (b, 0, 0)),
            pl.BlockSpec((_COUT, _C1 + _C2), lambda b, n: (0, 0)),
        ],
        out_specs=[
            pl.BlockSpec((1, _COUT, _NT), lambda b, n: (b, 0, n)),
            pl.BlockSpec((1, _COUT, 1), lambda b, n: (b, 0, 0)),
            pl.BlockSpec((1, _COUT, 1), lambda b, n: (b, 0, 0)),
        ],
        out_shape=[
            jax.ShapeDtypeStruct((_B, _COUT, _N), jnp.float32),
            jax.ShapeDtypeStruct((_B, _COUT, 1), jnp.float32),
            jax.ShapeDtypeStruct((_B, _COUT, 1), jnp.float32),
        ],
    )(uT, known, unknow_feats, known_feats, W0)

    out = pl.pallas_call(
        _bn_relu_kernel,
        grid=grid,
        in_specs=[
            pl.BlockSpec((1, _COUT, _NT), lambda b, n: (b, 0, n)),
            pl.BlockSpec((_B, _COUT, 1), lambda b, n: (0, 0, 0)),
            pl.BlockSpec((_B, _COUT, 1), lambda b, n: (0, 0, 0)),
            pl.BlockSpec((_COUT, 1), lambda b, n: (0, 0)),
            pl.BlockSpec((_COUT, 1), lambda b, n: (0, 0)),
        ],
        out_specs=pl.BlockSpec((1, _COUT, _NT), lambda b, n: (b, 0, n)),
        out_shape=jax.ShapeDtypeStruct((_B, _COUT, _N), jnp.float32),
    )(x, s, ss, gamma0.reshape(_COUT, 1), beta0.reshape(_COUT, 1))
    return out


# speedup vs baseline: 1.5400x; 1.1914x over previous
import jax
import jax.numpy as jnp
from jax.experimental import pallas as pl
from jax.experimental.pallas import tpu as pltpu

_B, _N, _M, _C1, _C2 = 4, 8192, 1024, 64, 128
_COUT = 128
_NT = 4096
_NB = _N // _NT


def _fp_fwd_kernel(uT_ref, known_ref, uf_ref, kf_ref,
                   w0_ref, x_ref, sum_ref, ssq_ref):
    uT = uT_ref[0]
    kn = known_ref[0]

    cross = jax.lax.dot(kn.astype(jnp.bfloat16), uT.astype(jnp.bfloat16),
                        preferred_element_type=jnp.float32)
    k2 = jnp.sum(kn * kn, axis=1, keepdims=True)
    u2 = jnp.sum(uT * uT, axis=0, keepdims=True)
    d2 = (u2 + k2) - 2.0 * cross

    inf = jnp.float32(jnp.inf)
    v1 = jnp.min(d2, axis=0, keepdims=True)
    c1 = d2 == v1
    w2 = jnp.where(c1, inf, d2)
    v2 = jnp.min(w2, axis=0, keepdims=True)
    c2 = w2 == v2
    w3 = jnp.where(c2, inf, w2)
    v3 = jnp.min(w3, axis=0, keepdims=True)
    c3 = w3 == v3

    r1, r2, r3 = [1.0 / (jnp.sqrt(jnp.maximum(v, 0.0)) + 1e-8)
                  for v in (v1, v2, v3)]
    inv_norm = 1.0 / (r1 + r2 + r3)
    S = jnp.where(c1, r1 * inv_norm,
                  jnp.where(c2, r2 * inv_norm,
                            jnp.where(c3, r3 * inv_norm, 0.0)))

    interp = jax.lax.dot(kf_ref[0], S, preferred_element_type=jnp.float32)
    w0 = w0_ref[...]
    x = (jax.lax.dot(w0[:, :_C2], interp, preferred_element_type=jnp.float32)
         + jax.lax.dot(w0[:, _C2:], uf_ref[0],
                       preferred_element_type=jnp.float32))
    x_ref[0] = x.astype(jnp.bfloat16)

    @pl.when(pl.program_id(1) == 0)
    def _init():
        sum_ref[...] = jnp.zeros_like(sum_ref)
        ssq_ref[...] = jnp.zeros_like(ssq_ref)

    sum_ref[0] += jnp.sum(x, axis=1, keepdims=True)
    ssq_ref[0] += jnp.sum(x * x, axis=1, keepdims=True)


def _bn_relu_kernel(x_ref, sum_ref, ssq_ref, g_ref, b_ref, o_ref):
    cnt = jnp.float32(_B * _N)
    mean = jnp.sum(sum_ref[...], axis=0) / cnt
    var = jnp.sum(ssq_ref[...], axis=0) / cnt - mean * mean
    scale = g_ref[...] * jax.lax.rsqrt(var + 1e-5)
    shift = b_ref[...] - mean * scale
    xf = x_ref[0].astype(jnp.float32)
    o_ref[0] = jnp.maximum(xf * scale + shift, 0.0)


def kernel(unknown, known, unknow_feats, known_feats, W0, gamma0, beta0):
    uT = jnp.transpose(unknown, (0, 2, 1))
    grid = (_B, _NB)
    x, s, ss = pl.pallas_call(
        _fp_fwd_kernel,
        grid=grid,
        in_specs=[
            pl.BlockSpec((1, 3, _NT), lambda b, n: (b, 0, n)),
            pl.BlockSpec((1, _M, 3), lambda b, n: (b, 0, 0)),
            pl.BlockSpec((1, _C1, _NT), lambda b, n: (b, 0, n)),
            pl.BlockSpec((1, _C2, _M), lambda b, n: (b, 0, 0)),
            pl.BlockSpec((_COUT, _C1 + _C2), lambda b, n: (0, 0)),
        ],
        out_specs=[
            pl.BlockSpec((1, _COUT, _NT), lambda b, n: (b, 0, n)),
            pl.BlockSpec((1, _COUT, 1), lambda b, n: (b, 0, 0)),
            pl.BlockSpec((1, _COUT, 1), lambda b, n: (b, 0, 0)),
        ],
        out_shape=[
            jax.ShapeDtypeStruct((_B, _COUT, _N), jnp.bfloat16),
            jax.ShapeDtypeStruct((_B, _COUT, 1), jnp.float32),
            jax.ShapeDtypeStruct((_B, _COUT, 1), jnp.float32),
        ],
    )(uT, known, unknow_feats, known_feats, W0)

    out = pl.pallas_call(
        _bn_relu_kernel,
        grid=grid,
        in_specs=[
            pl.BlockSpec((1, _COUT, _NT), lambda b, n: (b, 0, n)),
            pl.BlockSpec((_B, _COUT, 1), lambda b, n: (0, 0, 0)),
            pl.BlockSpec((_B, _COUT, 1), lambda b, n: (0, 0, 0)),
            pl.BlockSpec((_COUT, 1), lambda b, n: (0, 0)),
            pl.BlockSpec((_COUT, 1), lambda b, n: (0, 0)),
        ],
        out_specs=pl.BlockSpec((1, _COUT, _NT), lambda b, n: (b, 0, n)),
        out_shape=jax.ShapeDtypeStruct((_B, _COUT, _N), jnp.float32),
    )(x, s, ss, gamma0.reshape(_COUT, 1), beta0.reshape(_COUT, 1))
    return out
